# baseline (speedup 1.0000x reference)
import functools

import jax
import jax.numpy as jnp
from jax import lax
from jax.experimental import pallas as pl
from jax.experimental.pallas import tpu as pltpu
from jax.experimental.pallas import tpu_sc as plsc

_B, _N, _S, _K = 4, 16384, 1024, 64
_R2 = 0.1 * 0.1
_CIN = 32
_CMID, _COUT = 32, 64
_EPS = 1e-5
_G = 8
_SUB = 8
_NL = _N // _SUB

_INTERPRET = False


def _fps_body(xyz_ref, centers_ref, dists_ref):
    x = xyz_ref[:, 0, :, :]
    y = xyz_ref[:, 1, :, :]
    z = xyz_ref[:, 2, :, :]
    fi = (lax.broadcasted_iota(jnp.int32, (_B, _SUB, _NL), 1) * _NL
          + lax.broadcasted_iota(jnp.int32, (_B, _SUB, _NL), 2))
    dists_ref[...] = jnp.full((_B, _SUB, _NL), 1e10, jnp.float32)

    def _coords_at(idx):
        sel = fi == idx[:, None, None]
        lx = jnp.sum(jnp.where(sel, x, 0.0), axis=(1, 2))
        ly = jnp.sum(jnp.where(sel, y, 0.0), axis=(1, 2))
        lz = jnp.sum(jnp.where(sel, z, 0.0), axis=(1, 2))
        return lx, ly, lz

    def _store(i, lx, ly, lz):
        row = jnp.concatenate(
            [lx.reshape(1, 1, _B), ly.reshape(1, 1, _B), lz.reshape(1, 1, _B)],
            axis=1)
        centers_ref[pl.ds(i, 1), :, :] = row

    lx, ly, lz = _coords_at(jnp.zeros((_B,), jnp.int32))
    _store(0, lx, ly, lz)

    def step(i, carry):
        lx, ly, lz = carry
        dx = x - lx[:, None, None]
        dy = y - ly[:, None, None]
        dz = z - lz[:, None, None]
        d = (dx * dx + dy * dy) + dz * dz
        nd = jnp.minimum(dists_ref[...], d)
        dists_ref[...] = nd
        m = jnp.max(nd, axis=(1, 2))
        cand = jnp.where(nd == m[:, None, None], fi, _N)
        idx = jnp.min(cand, axis=(1, 2))
        lx, ly, lz = _coords_at(idx)
        _store(i, lx, ly, lz)
        return lx, ly, lz

    lax.fori_loop(1, _S, step, (lx, ly, lz))


def _fps(coords):
    xyz = coords.reshape(_B, 3, _SUB, _NL)
    return pl.pallas_call(
        _fps_body,
        out_shape=jax.ShapeDtypeStruct((_S, 3, _B), jnp.float32),
        scratch_shapes=[pltpu.VMEM((_B, _SUB, _NL), jnp.float32)],
        interpret=_INTERPRET,
    )(xyz)


def _pre_body(xin_ref, w1_ref, b1_ref, centers_ref,
              z_ref, pts4_ref, ctr4_ref, zc_ref):
    w1t = w1_ref[...].T
    xb = xin_ref[0]
    zb = jax.lax.dot_general(
        xb, w1t, (((0,), (0,)), ((), ())),
        preferred_element_type=jnp.float32)
    z_ref[0] = zb + b1_ref[...]
    px = xb[0, :]
    py = xb[1, :]
    pz = xb[2, :]
    p2 = (px * px + py * py) + pz * pz
    pts4_ref[0] = jnp.concatenate(
        [px.reshape(1, _N), py.reshape(1, _N), pz.reshape(1, _N),
         p2.reshape(1, _N)], axis=0)
    cb = centers_ref[0]
    c2 = jnp.sum(cb * cb, axis=1, keepdims=True)
    ctr4_ref[0] = jnp.concatenate(
        [cb, c2, jnp.zeros((_S, 4), jnp.float32)], axis=1)
    zc_ref[0] = jax.lax.dot_general(
        cb, w1t[:3, :], (((1,), (0,)), ((), ())),
        preferred_element_type=jnp.float32)


def _precompute(features, coords, W1, b1, centers):
    xin = jnp.concatenate([coords, features], axis=1)
    centers = jnp.transpose(centers, (2, 0, 1))
    return pl.pallas_call(
        _pre_body,
        out_shape=(
            jax.ShapeDtypeStruct((_B, _N, _CMID), jnp.float32),
            jax.ShapeDtypeStruct((_B, 4, _N), jnp.float32),
            jax.ShapeDtypeStruct((_B, _S, 8), jnp.float32),
            jax.ShapeDtypeStruct((_B, _S, _CMID), jnp.float32),
        ),
        grid=(_B,),
        in_specs=[
            pl.BlockSpec((1, 3 + _CIN, _N), lambda b: (b, 0, 0)),
            pl.BlockSpec((_CMID, 3 + _CIN), lambda b: (0, 0)),
            pl.BlockSpec((1, _CMID), lambda b: (0, 0)),
            pl.BlockSpec((1, _S, 3), lambda b: (b, 0, 0)),
        ],
        out_specs=(
            pl.BlockSpec((1, _N, _CMID), lambda b: (b, 0, 0)),
            pl.BlockSpec((1, 4, _N), lambda b: (b, 0, 0)),
            pl.BlockSpec((1, _S, 8), lambda b: (b, 0, 0)),
            pl.BlockSpec((1, _S, _CMID), lambda b: (b, 0, 0)),
        ),
        interpret=_INTERPRET,
    )(xin, W1, b1.reshape(1, _CMID), centers)


_NTILES = 32
_CPT = (_B * _S) // _NTILES
_TPB = _NTILES // _B
_GRP = 64
_NGRP = _N // _GRP


def _bf16r(x):
    u = plsc.bitcast(x, jnp.uint32)
    r = (u + jnp.uint32(0x7FFF) + ((u >> 16) & jnp.uint32(1))) \
        & jnp.uint32(0xFFFF0000)
    return plsc.bitcast(r, jnp.float32)


def _bq_body(pts4_hbm, ctr_hbm, z_hbm, out_hbm,
             px_v, py_v, pz_v, p2_v, ctr_v, stage_v, cidx_v, rows_v, sem):
    cid = lax.axis_index("c")
    sid = lax.axis_index("s")
    wid = sid * 2 + cid
    b = wid // _TPB
    pltpu.sync_copy(pts4_hbm.at[b, 0], px_v)
    pltpu.sync_copy(pts4_hbm.at[b, 1], py_v)
    pltpu.sync_copy(pts4_hbm.at[b, 2], pz_v)
    pltpu.sync_copy(pts4_hbm.at[b, 3], p2_v)
    pltpu.sync_copy(ctr_hbm.at[pl.ds(wid * (_CPT * 8), _CPT * 8)],
                    ctr_v.at[pl.ds(0, _CPT * 8)])
    lane = lax.iota(jnp.int32, 16)
    boff = b * _N

    def round_pts(i, _):
        o = i * 16
        px_v[pl.ds(o, 16)] = _bf16r(px_v[pl.ds(o, 16)])
        py_v[pl.ds(o, 16)] = _bf16r(py_v[pl.ds(o, 16)])
        pz_v[pl.ds(o, 16)] = _bf16r(pz_v[pl.ds(o, 16)])
        return 0

    lax.fori_loop(0, _N // 16, round_pts, 0)

    def one_center(c, _):
        idx8c = jnp.full((16,), 8 * c, jnp.int32)
        cxv = _bf16r(plsc.load_gather(ctr_v, [idx8c]))
        cyv = _bf16r(plsc.load_gather(ctr_v, [idx8c + 1]))
        czv = _bf16r(plsc.load_gather(ctr_v, [idx8c + 2]))
        c2v = plsc.load_gather(ctr_v, [idx8c + 3])

        def grp(g, cnt_v):
            base = g * _GRP
            for j in range(_GRP // 16):
                off = base + 16 * j
                px = px_v[pl.ds(off, 16)]
                py = py_v[pl.ds(off, 16)]
                pz = pz_v[pl.ds(off, 16)]
                p2 = p2_v[pl.ds(off, 16)]
                dot = cxv * px + cyv * py + czv * pz
                d2 = (c2v + p2) - 2.0 * dot
                m = d2 < _R2
                mi = m.astype(jnp.int32)
                cs = plsc.cumsum(mi)
                keep = jnp.logical_and(m, (cs + cnt_v) <= _K)
                idxv = off + lane
                plsc.store_scatter(stage_v, [cs + (cnt_v - 1)], idxv,
                                   mask=keep)
                pc = plsc.all_reduce_population_count(m)
                cnt_v = jnp.minimum(cnt_v + pc, _K)
            return cnt_v

        cnt_v = lax.fori_loop(0, _NGRP, grp, jnp.zeros((16,), jnp.int32))
        v0 = stage_v[pl.ds(0, 16)]
        first_v = plsc.cummax(jnp.where(lane == 0, v0, -1))
        padv = jnp.where(cnt_v == 0, 0, first_v)
        r = c // 2
        colb = (c % 2) * _K
        for t in range(_K // 16):
            v = stage_v[pl.ds(16 * t, 16)]
            pos = 16 * t + lane
            v = jnp.where(pos < cnt_v, v, padv) + boff
            cidx_v[r, pl.ds(colb + 16 * t, 16)] = v
        return 0

    lax.fori_loop(0, _CPT, one_center, 0)

    obase = wid * (_CPT * _K)
    nchunk = _CPT * _K // 128
    handles = [None, None]
    for chunk in range(nchunk + 1):
        if chunk < nchunk:
            handles[chunk % 2] = pltpu.async_copy(
                z_hbm.at[cidx_v.at[chunk]], rows_v.at[chunk % 2],
                sem.at[chunk % 2])
        if chunk >= 1:
            prev = chunk - 1
            handles[prev % 2].wait()
            pltpu.sync_copy(rows_v.at[prev % 2],
                            out_hbm.at[pl.ds(obase + prev * 128, 128), :])


def _ball_gather_sc(z, pts4, ctr8):
    mesh = plsc.VectorSubcoreMesh(core_axis_name="c", subcore_axis_name="s")
    f = functools.partial(
        pl.kernel,
        mesh=mesh,
        out_type=jax.ShapeDtypeStruct((_B * _S * _K, _CMID), jnp.float32),
        scratch_types=[
            pltpu.VMEM((_N,), jnp.float32),
            pltpu.VMEM((_N,), jnp.float32),
            pltpu.VMEM((_N,), jnp.float32),
            pltpu.VMEM((_N,), jnp.float32),
            pltpu.VMEM((_CPT * 8 + 16,), jnp.float32),
            pltpu.VMEM((96,), jnp.int32),
            pltpu.VMEM((_CPT * _K // 128, 128), jnp.int32),
            pltpu.VMEM((2, 128, _CMID), jnp.float32),
            pltpu.SemaphoreType.DMA((2,)),
        ],
        compiler_params=pltpu.CompilerParams(
            needs_layout_passes=False, use_tc_tiling_on_sc=False),
    )(_bq_body)
    return f(pts4, ctr8.reshape(_B * _S * 8), z.reshape(_B * _N, _CMID))


def _ball_gather_xla(z, pts4, ctr4):
    p2 = pts4[:, 3, :]
    c2 = ctr4[:, :, 3]
    dot = jnp.einsum('bsd,bdn->bsn', ctr4[:, :, :3], pts4[:, :3, :])
    d2 = c2[:, :, None] + p2[:, None, :] - 2.0 * dot
    mask = d2 < _R2
    cand = jnp.where(mask, jnp.arange(_N)[None, None, :], _N)
    neg_top, _ = lax.top_k(-cand, _K)
    cand_k = -neg_top
    first = cand_k[:, :, :1]
    first = jnp.where(first == _N, 0, first)
    idx = jnp.where(cand_k == _N, first, cand_k)
    flat = idx + jnp.arange(_B, dtype=jnp.int32)[:, None, None] * _N
    gathered = z.reshape(_B * _N, _CMID)[flat.reshape(-1)]
    return gathered


_CH = 128
_NCH = _S // _CH
_CHK = _CH * _K


def _tail_body(g_ref, zc_ref, g1_ref, be1_ref, w2_ref, b2_ref, g2_ref,
               be2_ref, out_ref, xbuf_ref, h2_ref, sem):
    b = pl.program_id(0)

    def load_chunk(c):
        cp = pltpu.make_async_copy(
            g_ref.at[b, pl.ds(c * _CHK, _CHK), :], xbuf_ref, sem)
        cp.start()
        cp.wait()
        xc = xbuf_ref[...].reshape(_CH, _K, _CMID)
        zcc = zc_ref[0, pl.ds(c * _CH, _CH), :]
        return xc - zcc[:, None, :]

    s1 = jnp.zeros((_CMID,), jnp.float32)
    ss1 = jnp.zeros((_CMID,), jnp.float32)
    for c in range(_NCH):
        x = load_chunk(c)
        s1 = s1 + jnp.sum(x, axis=(0, 1))
        ss1 = ss1 + jnp.sum(x * x, axis=(0, 1))
    n1 = float(_S * _K * (_CMID // _G))
    gid1 = jnp.arange(_CMID) // (_CMID // _G)
    a1 = (gid1[:, None] == gid1[None, :]).astype(jnp.float32)
    mean1 = (s1 @ a1) / n1
    var1 = (ss1 @ a1) / n1 - mean1 * mean1
    inv1 = lax.rsqrt(var1 + _EPS) * g1_ref[0]

    s2 = jnp.zeros((_COUT,), jnp.float32)
    ss2 = jnp.zeros((_COUT,), jnp.float32)
    for c in range(_NCH):
        x = load_chunk(c).reshape(_CHK, _CMID)
        xt = x.T
        h = (xt - mean1[:, None]) * inv1[:, None] + be1_ref[0][:, None]
        h = h * jax.nn.sigmoid(h)
        h2 = jax.lax.dot_general(
            w2_ref[...], h, (((1,), (0,)), ((), ())),
            preferred_element_type=jnp.float32) + b2_ref[0][:, None]
        h2_ref[:, pl.ds(c * _CHK, _CHK)] = h2
        s2 = s2 + jnp.sum(h2, axis=1)
        ss2 = ss2 + jnp.sum(h2 * h2, axis=1)
    n2 = float(_S * _K * (_COUT // _G))
    gid2 = jnp.arange(_COUT) // (_COUT // _G)
    a2 = (gid2[:, None] == gid2[None, :]).astype(jnp.float32)
    mean2 = (s2 @ a2) / n2
    var2 = (ss2 @ a2) / n2 - mean2 * mean2
    inv2 = lax.rsqrt(var2 + _EPS) * g2_ref[0]

    for c in range(_NCH):
        h2 = h2_ref[:, pl.ds(c * _CHK, _CHK)]
        hn = (h2 - mean2[:, None]) * inv2[:, None] + be2_ref[0][:, None]
        hn = hn * jax.nn.sigmoid(hn)
        pooled = jnp.max(hn.reshape(_COUT, _CH, _K), axis=2)
        out_ref[0, :, pl.ds(c * _CH, _CH)] = pooled


def _tail(gathered, zc, g1, be1, W2, b2, g2, be2):
    g = gathered.reshape(_B, _S * _K, _CMID)
    return pl.pallas_call(
        _tail_body,
        out_shape=jax.ShapeDtypeStruct((_B, _COUT, _S), jnp.float32),
        grid=(_B,),
        in_specs=[
            pl.BlockSpec(memory_space=pl.ANY),
            pl.BlockSpec((1, _S, _CMID), lambda b: (b, 0, 0)),
            pl.BlockSpec((1, _CMID), lambda b: (0, 0)),
            pl.BlockSpec((1, _CMID), lambda b: (0, 0)),
            pl.BlockSpec((_COUT, _CMID), lambda b: (0, 0)),
            pl.BlockSpec((1, _COUT), lambda b: (0, 0)),
            pl.BlockSpec((1, _COUT), lambda b: (0, 0)),
            pl.BlockSpec((1, _COUT), lambda b: (0, 0)),
        ],
        out_specs=pl.BlockSpec((1, _COUT, _S), lambda b: (b, 0, 0)),
        scratch_shapes=[
            pltpu.VMEM((_CHK, _CMID), jnp.float32),
            pltpu.VMEM((_COUT, _S * _K), jnp.float32),
            pltpu.SemaphoreType.DMA,
        ],
        interpret=_INTERPRET,
    )(g, zc, g1.reshape(1, _CMID), be1.reshape(1, _CMID), W2,
      b2.reshape(1, _COUT), g2.reshape(1, _COUT), be2.reshape(1, _COUT))


def kernel(features, coords, time_emb, W1, b1, g1, be1, W2, b2, g2, be2):
    centers = _fps(coords)
    z, pts4, ctr8, zc = _precompute(features, coords, W1, b1, centers)
    gathered = _ball_gather_sc(z, pts4, ctr8)
    out = _tail(gathered, zc, g1, be1, W2, b2, g2, be2)
    centers_coords = jnp.transpose(centers, (2, 1, 0))
    return (out, centers_coords, time_emb[:, :, :_S])


# speedup vs baseline: 1.4730x; 1.4730x over previous
import functools

import jax
import jax.numpy as jnp
from jax import lax
from jax.experimental import pallas as pl
from jax.experimental.pallas import tpu as pltpu
from jax.experimental.pallas import tpu_sc as plsc

_B, _N, _S, _K = 4, 16384, 1024, 64
_R2 = 0.1 * 0.1
_CIN = 32
_CMID, _COUT = 32, 64
_EPS = 1e-5
_G = 8
_SUB = 8
_NL = _N // _SUB

_INTERPRET = False


def _fps_body(xyz_ref, centers_ref, dists_ref):
    x = xyz_ref[:, 0, :, :]
    y = xyz_ref[:, 1, :, :]
    z = xyz_ref[:, 2, :, :]
    fi = (lax.broadcasted_iota(jnp.int32, (_B, _SUB, _NL), 1) * _NL
          + lax.broadcasted_iota(jnp.int32, (_B, _SUB, _NL), 2))
    dists_ref[...] = jnp.full((_B, _SUB, _NL), 1e10, jnp.float32)

    def _coords_at(idx):
        sel = fi == idx[:, None, None]
        lx = jnp.sum(jnp.where(sel, x, 0.0), axis=(1, 2))
        ly = jnp.sum(jnp.where(sel, y, 0.0), axis=(1, 2))
        lz = jnp.sum(jnp.where(sel, z, 0.0), axis=(1, 2))
        return lx, ly, lz

    def _store(i, lx, ly, lz):
        row = jnp.concatenate(
            [lx.reshape(1, 1, _B), ly.reshape(1, 1, _B), lz.reshape(1, 1, _B)],
            axis=1)
        centers_ref[pl.ds(i, 1), :, :] = row

    lx, ly, lz = _coords_at(jnp.zeros((_B,), jnp.int32))
    _store(0, lx, ly, lz)

    def step(i, carry):
        lx, ly, lz = carry
        dx = x - lx[:, None, None]
        dy = y - ly[:, None, None]
        dz = z - lz[:, None, None]
        d = (dx * dx + dy * dy) + dz * dz
        nd = jnp.minimum(dists_ref[...], d)
        dists_ref[...] = nd
        m = jnp.max(nd, axis=(1, 2))
        cand = jnp.where(nd == m[:, None, None], fi, _N)
        idx = jnp.min(cand, axis=(1, 2))
        lx, ly, lz = _coords_at(idx)
        _store(i, lx, ly, lz)
        return lx, ly, lz

    lax.fori_loop(1, _S, step, (lx, ly, lz))


def _fps(coords):
    xyz = coords.reshape(_B, 3, _SUB, _NL)
    return pl.pallas_call(
        _fps_body,
        out_shape=jax.ShapeDtypeStruct((_S, 3, _B), jnp.float32),
        scratch_shapes=[pltpu.VMEM((_B, _SUB, _NL), jnp.float32)],
        interpret=_INTERPRET,
    )(xyz)


def _pre_body(xin_ref, w1_ref, b1_ref, centers_ref,
              z_ref, pts4_ref, ctr4_ref, zc_ref):
    w1t = w1_ref[...].T
    xb = xin_ref[0]
    zb = jax.lax.dot_general(
        xb, w1t, (((0,), (0,)), ((), ())),
        preferred_element_type=jnp.float32)
    z_ref[0] = zb + b1_ref[...]
    px = xb[0, :]
    py = xb[1, :]
    pz = xb[2, :]
    p2 = (px * px + py * py) + pz * pz
    pts4_ref[0] = jnp.concatenate(
        [px.reshape(1, _N), py.reshape(1, _N), pz.reshape(1, _N),
         p2.reshape(1, _N)], axis=0)
    cb = centers_ref[0]
    c2 = jnp.sum(cb * cb, axis=1, keepdims=True)
    ctr4_ref[0] = jnp.concatenate(
        [cb, c2, jnp.zeros((_S, 4), jnp.float32)], axis=1)
    zc_ref[0] = jax.lax.dot_general(
        cb, w1t[:3, :], (((1,), (0,)), ((), ())),
        preferred_element_type=jnp.float32)


def _precompute(features, coords, W1, b1, centers):
    xin = jnp.concatenate([coords, features], axis=1)
    centers = jnp.transpose(centers, (2, 0, 1))
    return pl.pallas_call(
        _pre_body,
        out_shape=(
            jax.ShapeDtypeStruct((_B, _N, _CMID), jnp.float32),
            jax.ShapeDtypeStruct((_B, 4, _N), jnp.float32),
            jax.ShapeDtypeStruct((_B, _S, 8), jnp.float32),
            jax.ShapeDtypeStruct((_B, _S, _CMID), jnp.float32),
        ),
        grid=(_B,),
        in_specs=[
            pl.BlockSpec((1, 3 + _CIN, _N), lambda b: (b, 0, 0)),
            pl.BlockSpec((_CMID, 3 + _CIN), lambda b: (0, 0)),
            pl.BlockSpec((1, _CMID), lambda b: (0, 0)),
            pl.BlockSpec((1, _S, 3), lambda b: (b, 0, 0)),
        ],
        out_specs=(
            pl.BlockSpec((1, _N, _CMID), lambda b: (b, 0, 0)),
            pl.BlockSpec((1, 4, _N), lambda b: (b, 0, 0)),
            pl.BlockSpec((1, _S, 8), lambda b: (b, 0, 0)),
            pl.BlockSpec((1, _S, _CMID), lambda b: (b, 0, 0)),
        ),
        interpret=_INTERPRET,
    )(xin, W1, b1.reshape(1, _CMID), centers)


_NTILES = 32
_CPT = (_B * _S) // _NTILES
_TPB = _NTILES // _B
_GRP = 64
_NGRP = _N // _GRP


def _bf16r(x):
    u = plsc.bitcast(x, jnp.uint32)
    r = (u + jnp.uint32(0x7FFF) + ((u >> 16) & jnp.uint32(1))) \
        & jnp.uint32(0xFFFF0000)
    return plsc.bitcast(r, jnp.float32)


def _bq_body(pts4_hbm, ctr_hbm, z_hbm, out_hbm,
             px_v, py_v, pz_v, p2_v, ctr_v, stage_v, cidx_v, rows_v, sem):
    cid = lax.axis_index("c")
    sid = lax.axis_index("s")
    wid = sid * 2 + cid
    b = wid // _TPB
    pltpu.sync_copy(pts4_hbm.at[b, 0], px_v)
    pltpu.sync_copy(pts4_hbm.at[b, 1], py_v)
    pltpu.sync_copy(pts4_hbm.at[b, 2], pz_v)
    pltpu.sync_copy(pts4_hbm.at[b, 3], p2_v)
    pltpu.sync_copy(ctr_hbm.at[pl.ds(wid * (_CPT * 8), _CPT * 8)],
                    ctr_v.at[pl.ds(0, _CPT * 8)])
    lane = lax.iota(jnp.int32, 16)
    boff = b * _N

    def round_pts(i, _):
        o = i * 16
        px_v[pl.ds(o, 16)] = _bf16r(px_v[pl.ds(o, 16)])
        py_v[pl.ds(o, 16)] = _bf16r(py_v[pl.ds(o, 16)])
        pz_v[pl.ds(o, 16)] = _bf16r(pz_v[pl.ds(o, 16)])
        return 0

    lax.fori_loop(0, _N // 16, round_pts, 0)

    def one_center(c, _):
        idx8c = jnp.full((16,), 8 * c, jnp.int32)
        cxv = _bf16r(plsc.load_gather(ctr_v, [idx8c]))
        cyv = _bf16r(plsc.load_gather(ctr_v, [idx8c + 1]))
        czv = _bf16r(plsc.load_gather(ctr_v, [idx8c + 2]))
        c2v = plsc.load_gather(ctr_v, [idx8c + 3])

        def grp(g, carry):
            base = g * _GRP
            ms = []
            tot = jnp.zeros((16,), jnp.int32)
            for j in range(_GRP // 16):
                off = base + 16 * j
                px = px_v[pl.ds(off, 16)]
                py = py_v[pl.ds(off, 16)]
                pz = pz_v[pl.ds(off, 16)]
                p2 = p2_v[pl.ds(off, 16)]
                dot = cxv * px + cyv * py + czv * pz
                d2 = (c2v + p2) - 2.0 * dot
                m = d2 < _R2
                ms.append(m)
                tot = tot + m.astype(jnp.int32)
            npc = jnp.sum(tot)

            def with_hits(carry):
                cnt, first = carry
                for j in range(_GRP // 16):
                    m = ms[j]
                    mi = m.astype(jnp.int32)
                    pcj = jnp.sum(mi)
                    cs = plsc.cumsum(mi)
                    keep = jnp.logical_and(m, (cs + cnt) <= _K)
                    idxv = base + 16 * j + lane
                    fmin = jnp.min(idxv + (1 - mi) * _N)
                    first = jnp.where((cnt == 0) & (pcj > 0), fmin, first)
                    plsc.store_scatter(stage_v, [cs + (cnt - 1)], idxv,
                                       mask=keep)
                    cnt = cnt + jnp.minimum(pcj, _K - cnt)
                return cnt, first

            return lax.cond(npc > 0, with_hits, lambda cf: cf, carry)

        cnt, first = lax.fori_loop(0, _NGRP, grp,
                                   (jnp.int32(0), jnp.int32(0)))
        r = c // 2
        colb = (c % 2) * _K
        for t in range(_K // 16):
            v = stage_v[pl.ds(16 * t, 16)]
            pos = 16 * t + lane
            v = jnp.where(pos < cnt, v, first) + boff
            cidx_v[r, pl.ds(colb + 16 * t, 16)] = v
        return 0

    lax.fori_loop(0, _CPT, one_center, 0)

    obase = wid * (_CPT * _K)
    nchunk = _CPT * _K // 128
    handles = [None, None]
    for chunk in range(nchunk + 1):
        if chunk < nchunk:
            handles[chunk % 2] = pltpu.async_copy(
                z_hbm.at[cidx_v.at[chunk]], rows_v.at[chunk % 2],
                sem.at[chunk % 2])
        if chunk >= 1:
            prev = chunk - 1
            handles[prev % 2].wait()
            pltpu.sync_copy(rows_v.at[prev % 2],
                            out_hbm.at[pl.ds(obase + prev * 128, 128), :])


def _ball_gather_sc(z, pts4, ctr8):
    mesh = plsc.VectorSubcoreMesh(core_axis_name="c", subcore_axis_name="s")
    f = functools.partial(
        pl.kernel,
        mesh=mesh,
        out_type=jax.ShapeDtypeStruct((_B * _S * _K, _CMID), jnp.float32),
        scratch_types=[
            pltpu.VMEM((_N,), jnp.float32),
            pltpu.VMEM((_N,), jnp.float32),
            pltpu.VMEM((_N,), jnp.float32),
            pltpu.VMEM((_N,), jnp.float32),
            pltpu.VMEM((_CPT * 8 + 16,), jnp.float32),
            pltpu.VMEM((96,), jnp.int32),
            pltpu.VMEM((_CPT * _K // 128, 128), jnp.int32),
            pltpu.VMEM((2, 128, _CMID), jnp.float32),
            pltpu.SemaphoreType.DMA((2,)),
        ],
        compiler_params=pltpu.CompilerParams(
            needs_layout_passes=False, use_tc_tiling_on_sc=False),
    )(_bq_body)
    return f(pts4, ctr8.reshape(_B * _S * 8), z.reshape(_B * _N, _CMID))


_CH = 128
_NCH = _S // _CH
_CHK = _CH * _K


def _tail_body(g_ref, zc_ref, g1_ref, be1_ref, w2_ref, b2_ref, g2_ref,
               be2_ref, out_ref, xbuf_ref, h2_ref, sem):
    b = pl.program_id(0)

    def load_chunk(c):
        cp = pltpu.make_async_copy(
            g_ref.at[b, pl.ds(c * _CHK, _CHK), :], xbuf_ref, sem)
        cp.start()
        cp.wait()
        xc = xbuf_ref[...].reshape(_CH, _K, _CMID)
        zcc = zc_ref[0, pl.ds(c * _CH, _CH), :]
        return xc - zcc[:, None, :]

    s1 = jnp.zeros((_CMID,), jnp.float32)
    ss1 = jnp.zeros((_CMID,), jnp.float32)
    for c in range(_NCH):
        x = load_chunk(c)
        s1 = s1 + jnp.sum(x, axis=(0, 1))
        ss1 = ss1 + jnp.sum(x * x, axis=(0, 1))
    n1 = float(_S * _K * (_CMID // _G))
    gid1 = jnp.arange(_CMID) // (_CMID // _G)
    a1 = (gid1[:, None] == gid1[None, :]).astype(jnp.float32)
    mean1 = (s1 @ a1) / n1
    var1 = (ss1 @ a1) / n1 - mean1 * mean1
    inv1 = lax.rsqrt(var1 + _EPS) * g1_ref[0]

    s2 = jnp.zeros((_COUT,), jnp.float32)
    ss2 = jnp.zeros((_COUT,), jnp.float32)
    for c in range(_NCH):
        x = load_chunk(c).reshape(_CHK, _CMID)
        xt = x.T
        h = (xt - mean1[:, None]) * inv1[:, None] + be1_ref[0][:, None]
        h = h * jax.nn.sigmoid(h)
        h2 = jax.lax.dot_general(
            w2_ref[...], h, (((1,), (0,)), ((), ())),
            preferred_element_type=jnp.float32) + b2_ref[0][:, None]
        h2_ref[:, pl.ds(c * _CHK, _CHK)] = h2
        s2 = s2 + jnp.sum(h2, axis=1)
        ss2 = ss2 + jnp.sum(h2 * h2, axis=1)
    n2 = float(_S * _K * (_COUT // _G))
    gid2 = jnp.arange(_COUT) // (_COUT // _G)
    a2 = (gid2[:, None] == gid2[None, :]).astype(jnp.float32)
    mean2 = (s2 @ a2) / n2
    var2 = (ss2 @ a2) / n2 - mean2 * mean2
    inv2 = lax.rsqrt(var2 + _EPS) * g2_ref[0]

    for c in range(_NCH):
        h2 = h2_ref[:, pl.ds(c * _CHK, _CHK)]
        hn = (h2 - mean2[:, None]) * inv2[:, None] + be2_ref[0][:, None]
        hn = hn * jax.nn.sigmoid(hn)
        pooled = jnp.max(hn.reshape(_COUT, _CH, _K), axis=2)
        out_ref[0, :, pl.ds(c * _CH, _CH)] = pooled


def _tail(gathered, zc, g1, be1, W2, b2, g2, be2):
    g = gathered.reshape(_B, _S * _K, _CMID)
    return pl.pallas_call(
        _tail_body,
        out_shape=jax.ShapeDtypeStruct((_B, _COUT, _S), jnp.float32),
        grid=(_B,),
        in_specs=[
            pl.BlockSpec(memory_space=pl.ANY),
            pl.BlockSpec((1, _S, _CMID), lambda b: (b, 0, 0)),
            pl.BlockSpec((1, _CMID), lambda b: (0, 0)),
            pl.BlockSpec((1, _CMID), lambda b: (0, 0)),
            pl.BlockSpec((_COUT, _CMID), lambda b: (0, 0)),
            pl.BlockSpec((1, _COUT), lambda b: (0, 0)),
            pl.BlockSpec((1, _COUT), lambda b: (0, 0)),
            pl.BlockSpec((1, _COUT), lambda b: (0, 0)),
        ],
        out_specs=pl.BlockSpec((1, _COUT, _S), lambda b: (b, 0, 0)),
        scratch_shapes=[
            pltpu.VMEM((_CHK, _CMID), jnp.float32),
            pltpu.VMEM((_COUT, _S * _K), jnp.float32),
            pltpu.SemaphoreType.DMA,
        ],
        interpret=_INTERPRET,
    )(g, zc, g1.reshape(1, _CMID), be1.reshape(1, _CMID), W2,
      b2.reshape(1, _COUT), g2.reshape(1, _COUT), be2.reshape(1, _COUT))


def kernel(features, coords, time_emb, W1, b1, g1, be1, W2, b2, g2, be2):
    centers = _fps(coords)
    z, pts4, ctr8, zc = _precompute(features, coords, W1, b1, centers)
    gathered = _ball_gather_sc(z, pts4, ctr8)
    out = _tail(gathered, zc, g1, be1, W2, b2, g2, be2)
    centers_coords = jnp.transpose(centers, (2, 1, 0))
    return (out, centers_coords, time_emb[:, :, :_S])
